# Initial kernel scaffold; baseline (speedup 1.0000x reference)
import jax
import jax.numpy as jnp
from jax.experimental import pallas as pl


def kernel(x, W):
    raise NotImplementedError("write your pallas kernel here")


# speedup vs baseline: 1.5597x; 1.5597x over previous
import functools

import jax
import jax.numpy as jnp
from jax import lax
from jax.experimental import pallas as pl
from jax.experimental.pallas import tpu as pltpu
from jax.experimental.pallas import tpu_sc as plsc

_info = plsc.get_sparse_core_info()
_NC, _NS = _info.num_cores, _info.num_subcores
_NW = _NC * _NS

_IDX_PER_GATHER = 128
_GATHERS_PER_CHUNK = 8
_CHUNK = _IDX_PER_GATHER * _GATHERS_PER_CHUNK


@functools.partial(jax.jit, static_argnums=(2, 3))
def _embed_gather(x_resh, W, b_per_w, n_chunks):
    D = W.shape[1]
    B = b_per_w * _NW
    n_gath = b_per_w // _IDX_PER_GATHER

    mesh = plsc.VectorSubcoreMesh(core_axis_name="c", subcore_axis_name="s")

    @functools.partial(
        pl.kernel,
        mesh=mesh,
        out_type=jax.ShapeDtypeStruct((B, D), jnp.float32),
        scratch_types=[
            pltpu.VMEM((n_gath, _IDX_PER_GATHER), jnp.int32),
            pltpu.VMEM((_CHUNK, D), jnp.float32),
            pltpu.SemaphoreType.DMA,
        ],
        compiler_params=pltpu.CompilerParams(use_tc_tiling_on_sc=False),
    )
    def body(x_hbm, w_hbm, out_hbm, idx_v, rows_v, sem):
        wid = lax.axis_index("s") * _NC + lax.axis_index("c")
        base = wid * b_per_w
        pltpu.sync_copy(x_hbm.at[wid], idx_v)

        def chunk_body(c, carry):
            handles = []
            for j in range(_GATHERS_PER_CHUNK):
                handles.append(
                    pltpu.async_copy(
                        w_hbm.at[idx_v.at[c * _GATHERS_PER_CHUNK + j]],
                        rows_v.at[pl.ds(j * _IDX_PER_GATHER, _IDX_PER_GATHER)],
                        sem,
                    )
                )
            for h in handles:
                h.wait()
            pltpu.sync_copy(rows_v, out_hbm.at[pl.ds(base + c * _CHUNK, _CHUNK)])
            return carry

        lax.fori_loop(0, n_chunks, chunk_body, 0)

    return body(x_resh, W)


def kernel(x, W):
    orig_shape = x.shape
    D = W.shape[1]
    B = x.size
    b_per_w = B // _NW
    n_chunks = b_per_w // _CHUNK
    n_gath = b_per_w // _IDX_PER_GATHER
    x_resh = x.reshape(_NW, n_gath, _IDX_PER_GATHER)
    out = _embed_gather(x_resh, W, b_per_w, n_chunks)
    return out.reshape(*orig_shape, D)


# speedup vs baseline: 1.5761x; 1.0105x over previous
import functools

import jax
import jax.numpy as jnp
from jax import lax
from jax.experimental import pallas as pl
from jax.experimental.pallas import tpu as pltpu
from jax.experimental.pallas import tpu_sc as plsc

_info = plsc.get_sparse_core_info()
_NC, _NS = _info.num_cores, _info.num_subcores
_NW = _NC * _NS

_IDX_PER_GATHER = 128
_GATHERS_PER_CHUNK = 8
_CHUNK = _IDX_PER_GATHER * _GATHERS_PER_CHUNK


@functools.partial(jax.jit, static_argnums=(2, 3))
def _embed_gather(x_resh, W, b_per_w, n_chunks):
    D = W.shape[1]
    B = b_per_w * _NW
    n_gath = b_per_w // _IDX_PER_GATHER

    mesh = plsc.VectorSubcoreMesh(core_axis_name="c", subcore_axis_name="s")

    @functools.partial(
        pl.kernel,
        mesh=mesh,
        out_type=jax.ShapeDtypeStruct((B, D), jnp.float32),
        scratch_types=[
            pltpu.VMEM((n_gath, _IDX_PER_GATHER), jnp.int32),
            pltpu.VMEM((2, _CHUNK, D), jnp.float32),
            pltpu.SemaphoreType.DMA,
        ],
        compiler_params=pltpu.CompilerParams(use_tc_tiling_on_sc=False),
    )
    def body(x_hbm, w_hbm, out_hbm, idx_v, rows_v, sem):
        wid = lax.axis_index("s") * _NC + lax.axis_index("c")
        base = wid * b_per_w
        pltpu.sync_copy(x_hbm.at[wid], idx_v)

        def issue_gathers(c, buf):
            for j in range(_GATHERS_PER_CHUNK):
                pltpu.async_copy(
                    w_hbm.at[idx_v.at[c * _GATHERS_PER_CHUNK + j]],
                    rows_v.at[buf].at[pl.ds(j * _IDX_PER_GATHER, _IDX_PER_GATHER)],
                    sem,
                )

        issue_gathers(0, 0)

        def chunk_body(c, carry):
            @pl.when(c + 1 < n_chunks)
            def _():
                issue_gathers(c + 1, (c + 1) % 2)

            pltpu.make_async_copy(
                w_hbm.at[pl.ds(0, _CHUNK)], rows_v.at[c % 2], sem
            ).wait()
            pltpu.sync_copy(
                rows_v.at[c % 2], out_hbm.at[pl.ds(base + c * _CHUNK, _CHUNK)]
            )
            return carry

        lax.fori_loop(0, n_chunks, chunk_body, 0)

    return body(x_resh, W)


def kernel(x, W):
    orig_shape = x.shape
    D = W.shape[1]
    B = x.size
    b_per_w = B // _NW
    n_chunks = b_per_w // _CHUNK
    n_gath = b_per_w // _IDX_PER_GATHER
    x_resh = x.reshape(_NW, n_gath, _IDX_PER_GATHER)
    out = _embed_gather(x_resh, W, b_per_w, n_chunks)
    return out.reshape(*orig_shape, D)
